# baseline (speedup 1.0000x reference)
import functools

import jax
import jax.numpy as jnp
from jax.experimental import pallas as pl
from jax.experimental.pallas import tpu as pltpu
from jax.experimental.pallas import tpu_sc as plsc

_NEG = -3.0e38
_BIG = 1 << 30
_K = 8
_NC = 2
_NS = 16
_LANES = 16
_PW = 128


def _router_kernel(nb, x_ref, rw_ref, emb_ref,
                   idx_ref, c_ref, psum_ref,
                   l_scr, cv_scr, ci_scr, ca_scr):
    i = pl.program_id(0)
    j = pl.program_id(1)
    t = x_ref.shape[0]
    nblk = rw_ref.shape[1]

    x = x_ref[...]
    logits = jax.lax.dot_general(
        x, rw_ref[...], (((1,), (0,)), ((), ())),
        preferred_element_type=jnp.float32)
    act = jax.lax.dot_general(
        x, emb_ref[...], (((1,), (1,)), ((), ())),
        preferred_element_type=jnp.float32)
    l_scr[j] = logits

    iota = jax.lax.broadcasted_iota(jnp.int32, (t, nblk), 1) + j * nblk
    l = logits
    vals, idxs, acts = [], [], []
    for _ in range(_K):
        m = jnp.max(l, axis=1, keepdims=True)
        ik = jnp.min(jnp.where(l >= m, iota, _BIG), axis=1, keepdims=True)
        oh = iota == ik
        a_k = jnp.sum(jnp.where(oh, act, 0.0), axis=1, keepdims=True)
        vals.append(m)
        idxs.append(ik)
        acts.append(a_k)
        l = jnp.where(oh, _NEG, l)
    cv_scr[j] = jnp.concatenate(vals, axis=1)
    ci_scr[j] = jnp.concatenate(idxs, axis=1)
    ca_scr[j] = jnp.concatenate(acts, axis=1)

    @pl.when(j == nb - 1)
    def _finalize():
        iota8 = jax.lax.broadcasted_iota(jnp.int32, (t, _K), 1)
        mv = [cv_scr[jj] for jj in range(nb)]
        ci = [ci_scr[jj] for jj in range(nb)]
        ca = [ca_scr[jj] for jj in range(nb)]
        svals, sidx, sact = [], [], []
        for _ in range(_K):
            m = mv[0].max(axis=1, keepdims=True)
            for jj in range(1, nb):
                m = jnp.maximum(m, mv[jj].max(axis=1, keepdims=True))
            pcode = jnp.full((t, 1), _BIG, jnp.int32)
            for jj in range(nb):
                pj = jnp.min(jnp.where(mv[jj] >= m, iota8 + jj * _K, _BIG),
                             axis=1, keepdims=True)
                pcode = jnp.minimum(pcode, pj)
            iv = jnp.zeros((t, 1), jnp.int32)
            av = jnp.zeros((t, 1), jnp.float32)
            for jj in range(nb):
                oh = (iota8 + jj * _K) == pcode
                iv = iv + jnp.sum(jnp.where(oh, ci[jj], 0),
                                  axis=1, keepdims=True)
                av = av + jnp.sum(jnp.where(oh, ca[jj], 0.0),
                                  axis=1, keepdims=True)
                mv[jj] = jnp.where(oh, _NEG, mv[jj])
            svals.append(m)
            sidx.append(iv)
            sact.append(av)
        tkv = jnp.concatenate(svals, axis=1)
        tki = jnp.concatenate(sidx, axis=1)
        tka = jnp.concatenate(sact, axis=1)
        ge = jnp.exp(tkv - tkv[:, 0:1])
        gate = ge / jnp.sum(ge, axis=1, keepdims=True)
        idx_ref[...] = tki
        c_ref[...] = gate * tka

        m_row = tkv[:, 0:1]
        s_row = jnp.zeros((t, 1), jnp.float32)
        for jj in range(nb):
            e = jnp.exp(l_scr[jj] - m_row)
            l_scr[jj] = e
            s_row = s_row + jnp.sum(e, axis=1, keepdims=True)
        r_row = 1.0 / s_row

        @pl.when(i == 0)
        def _():
            psum_ref[...] = jnp.zeros_like(psum_ref)

        for jj in range(nb):
            chunk = jax.lax.dot_general(
                l_scr[jj], r_row, (((0,), (0,)), ((), ())),
                preferred_element_type=jnp.float32)
            psum_ref[pl.ds(jj * nblk, nblk), :] += jnp.broadcast_to(
                chunk, (nblk, _PW))


def _sc_emit_kernel(nch, cpt, d, w_ref, idx_ref, c_ref, p_ref,
                    out_ref, aux_ref,
                    idx_v, c_v, rows_v, out_v, pacc_v, aux_v, sem, sem2):
    wid = jax.lax.axis_index("s") * _NC + jax.lax.axis_index("c")
    tpw = nch * cpt
    dch = d // _LANES
    pltpu.sync_copy(idx_ref.at[wid], idx_v)
    pltpu.sync_copy(c_ref.at[wid], c_v)

    unroll = 4

    def body(cc, acc):
        pltpu.async_copy(w_ref.at[idx_v.at[cc]], rows_v, sem).wait()
        pltpu.async_copy(p_ref.at[idx_v.at[cc]], pacc_v, sem2).wait()

        def tok_body(t, carry2):
            cks = []
            for k in range(_K):
                pos = (cc * cpt + t) * _K + k
                cks.append(c_v[pl.ds(pos * _LANES, _LANES)])

            def dc_body(g, carry3):
                for u in range(unroll):
                    sl = pl.ds((g * unroll + u) * _LANES, _LANES)
                    a = cks[0] * rows_v[t * _K, sl]
                    for k in range(1, _K):
                        a = a + cks[k] * rows_v[t * _K + k, sl]
                    out_v[t, sl] = a
                return carry3

            return jax.lax.fori_loop(0, dch // unroll, dc_body, carry2)

        jax.lax.fori_loop(0, cpt, tok_body, 0)
        pltpu.sync_copy(out_v, out_ref.at[pl.ds(wid * tpw + cc * cpt, cpt)])
        for r in range(cpt * _K):
            acc = acc + pacc_v[r, pl.ds(0, _LANES)]
        return acc

    acc = jax.lax.fori_loop(0, nch, body, jnp.zeros((_LANES,), jnp.float32))
    aux_v[...] = acc
    pltpu.sync_copy(aux_v, aux_ref.at[wid])


def _emit_kernel(w_ref, idx_ref, c_ref, out_ref):
    j = pl.program_id(1)
    t = idx_ref.shape[0]
    nblk = w_ref.shape[0]
    iota = jax.lax.broadcasted_iota(jnp.int32, (t, nblk), 1) + j * nblk
    gated = jnp.zeros((t, nblk), jnp.float32)
    for k in range(_K):
        ik = idx_ref[:, k:k + 1]
        ck = c_ref[:, k:k + 1]
        gated = gated + jnp.where(iota == ik, ck, 0.0)
    partial = jax.lax.dot_general(
        gated, w_ref[...], (((1,), (0,)), ((), ())),
        preferred_element_type=jnp.float32)

    @pl.when(j == 0)
    def _():
        out_ref[...] = jnp.zeros_like(out_ref)
    out_ref[...] += partial


def kernel(x, know_emb, know_w, router_w, attention_mask):
    b, s, d = x.shape
    n = router_w.shape[1]
    tokens = b * s
    t_blk = min(512, tokens)
    nblk = min(1024, n)
    gi, gj = tokens // t_blk, n // nblk
    xf = x.reshape(tokens, d)

    idx, c, psum = pl.pallas_call(
        functools.partial(_router_kernel, gj),
        grid=(gi, gj),
        in_specs=[
            pl.BlockSpec((t_blk, d), lambda i, j: (i, 0)),
            pl.BlockSpec((d, nblk), lambda i, j: (0, j)),
            pl.BlockSpec((nblk, d), lambda i, j: (j, 0)),
        ],
        out_specs=[
            pl.BlockSpec((t_blk, _K), lambda i, j: (i, 0)),
            pl.BlockSpec((t_blk, _K), lambda i, j: (i, 0)),
            pl.BlockSpec((n, _PW), lambda i, j: (0, 0)),
        ],
        out_shape=[
            jax.ShapeDtypeStruct((tokens, _K), jnp.int32),
            jax.ShapeDtypeStruct((tokens, _K), jnp.float32),
            jax.ShapeDtypeStruct((n, _PW), jnp.float32),
        ],
        scratch_shapes=[
            pltpu.VMEM((gj, t_blk, nblk), jnp.float32),
            pltpu.VMEM((gj, t_blk, _K), jnp.float32),
            pltpu.VMEM((gj, t_blk, _K), jnp.int32),
            pltpu.VMEM((gj, t_blk, _K), jnp.float32),
        ],
    )(xf, router_w, know_emb)

    nw = _NC * _NS
    tpw = tokens // nw
    cpt = 4
    nch = tpw // cpt
    idx3 = idx.reshape(nw, nch, cpt * _K)
    c2 = jnp.broadcast_to(
        c.reshape(nw, tpw * _K, 1),
        (nw, tpw * _K, _LANES)).reshape(nw, tpw * _K * _LANES)
    mesh = plsc.VectorSubcoreMesh(core_axis_name="c", subcore_axis_name="s")
    out, aux_p = pl.kernel(
        functools.partial(_sc_emit_kernel, nch, cpt, d),
        mesh=mesh,
        out_type=[
            jax.ShapeDtypeStruct((tokens, d), jnp.float32),
            jax.ShapeDtypeStruct((nw, _LANES), jnp.float32),
        ],
        scratch_types=[
            pltpu.VMEM((nch, cpt * _K), jnp.int32),
            pltpu.VMEM((tpw * _K * _LANES,), jnp.float32),
            pltpu.VMEM((cpt * _K, d), jnp.float32),
            pltpu.VMEM((cpt, d), jnp.float32),
            pltpu.VMEM((cpt * _K, _PW), jnp.float32),
            pltpu.VMEM((_LANES,), jnp.float32),
            pltpu.SemaphoreType.DMA,
            pltpu.SemaphoreType.DMA,
        ],
    )(know_w, idx3, c2, psum)

    aux = (jnp.float32(n) / (jnp.float32(tokens) * jnp.float32(tokens * _K))
           ) * jnp.sum(aux_p[:, 0])
    return out.reshape(b, s, d), aux


# speedup vs baseline: 1.5372x; 1.5372x over previous
import functools

import jax
import jax.numpy as jnp
from jax.experimental import pallas as pl
from jax.experimental.pallas import tpu as pltpu
from jax.experimental.pallas import tpu_sc as plsc

_NEG = -3.0e38
_BIG = 1 << 30
_K = 8
_NC = 2
_NS = 16
_LANES = 16
_PW = 128


def _router_kernel(nb, x_ref, rw_ref, emb_ref,
                   idx_ref, c_ref, psum_ref,
                   l_scr, cv_scr, ci_scr, ca_scr):
    i = pl.program_id(0)
    j = pl.program_id(1)
    t = x_ref.shape[0]
    nblk = rw_ref.shape[1]

    x = x_ref[...]
    logits = jax.lax.dot_general(
        x, rw_ref[...], (((1,), (0,)), ((), ())),
        preferred_element_type=jnp.float32)
    act = jax.lax.dot_general(
        x, emb_ref[...], (((1,), (1,)), ((), ())),
        preferred_element_type=jnp.float32)
    l_scr[j] = logits

    iota = jax.lax.broadcasted_iota(jnp.int32, (t, nblk), 1) + j * nblk
    l = logits
    vals, idxs, acts = [], [], []
    for _ in range(_K):
        m = jnp.max(l, axis=1, keepdims=True)
        ik = jnp.min(jnp.where(l >= m, iota, _BIG), axis=1, keepdims=True)
        oh = iota == ik
        a_k = jnp.sum(jnp.where(oh, act, 0.0), axis=1, keepdims=True)
        vals.append(m)
        idxs.append(ik)
        acts.append(a_k)
        l = jnp.where(oh, _NEG, l)
    cv_scr[j] = jnp.concatenate(vals, axis=1)
    ci_scr[j] = jnp.concatenate(idxs, axis=1)
    ca_scr[j] = jnp.concatenate(acts, axis=1)

    @pl.when(j == nb - 1)
    def _finalize():
        iota64 = jax.lax.broadcasted_iota(jnp.int32, (t, nb * _K), 1)
        mv = jnp.concatenate([cv_scr[jj] for jj in range(nb)], axis=1)
        ci64 = jnp.concatenate([ci_scr[jj] for jj in range(nb)], axis=1)
        ca64 = jnp.concatenate([ca_scr[jj] for jj in range(nb)], axis=1)
        svals, sidx, sact = [], [], []
        for _ in range(_K):
            m = jnp.max(mv, axis=1, keepdims=True)
            pcode = jnp.min(jnp.where(mv >= m, iota64, _BIG),
                            axis=1, keepdims=True)
            oh = iota64 == pcode
            iv = jnp.sum(jnp.where(oh, ci64, 0), axis=1, keepdims=True)
            av = jnp.sum(jnp.where(oh, ca64, 0.0), axis=1, keepdims=True)
            mv = jnp.where(oh, _NEG, mv)
            svals.append(m)
            sidx.append(iv)
            sact.append(av)
        tkv = jnp.concatenate(svals, axis=1)
        tki = jnp.concatenate(sidx, axis=1)
        tka = jnp.concatenate(sact, axis=1)
        ge = jnp.exp(tkv - tkv[:, 0:1])
        gate = ge / jnp.sum(ge, axis=1, keepdims=True)
        idx_ref[...] = tki
        c_ref[...] = gate * tka

        m_row = tkv[:, 0:1]
        s_row = jnp.zeros((t, 1), jnp.float32)
        for jj in range(nb):
            e = jnp.exp(l_scr[jj] - m_row)
            l_scr[jj] = e
            s_row = s_row + jnp.sum(e, axis=1, keepdims=True)
        r_row = 1.0 / s_row

        @pl.when(i == 0)
        def _():
            psum_ref[...] = jnp.zeros_like(psum_ref)

        for jj in range(nb):
            chunk = jax.lax.dot_general(
                l_scr[jj], r_row, (((0,), (0,)), ((), ())),
                preferred_element_type=jnp.float32)
            psum_ref[pl.ds(jj * nblk, nblk), :] += jnp.broadcast_to(
                chunk, (nblk, _PW))


def _sc_emit_kernel(nch, cpt, d, w_ref, idx_ref, idxf_ref, c_ref, p_ref,
                    out_ref, aux_ref,
                    idx_v, idx_f, c_v, rows_a, rows_b, out_v, pacc_v, aux_v,
                    sem_a, sem_b, sem_p):
    wid = jax.lax.axis_index("s") * _NC + jax.lax.axis_index("c")
    tpw = nch * cpt
    dch = d // _LANES
    gsz = 2 * cpt * _K
    pltpu.sync_copy(idx_ref.at[wid], idx_v)
    pltpu.sync_copy(idxf_ref.at[wid], idx_f)
    pltpu.sync_copy(c_ref.at[wid], c_v)

    unroll = 4

    def compute_chunk(cc, rows_v):
        def tok_body(tt, carry2):
            cks = []
            for k in range(_K):
                pos = (cc * cpt + tt) * _K + k
                cks.append(c_v[pl.ds(pos * _LANES, _LANES)])

            def dc_body(g, carry3):
                for u in range(unroll):
                    sl = pl.ds((g * unroll + u) * _LANES, _LANES)
                    a = cks[0] * rows_v[tt * _K, sl]
                    for k in range(1, _K):
                        a = a + cks[k] * rows_v[tt * _K + k, sl]
                    out_v[tt, sl] = a
                return carry3

            return jax.lax.fori_loop(0, dch // unroll, dc_body, carry2)

        jax.lax.fori_loop(0, cpt, tok_body, 0)
        pltpu.sync_copy(out_v, out_ref.at[pl.ds(wid * tpw + cc * cpt, cpt)])

    pltpu.async_copy(w_ref.at[idx_v.at[0]], rows_a, sem_a)

    def body(g, acc):
        cc0 = 2 * g
        cc1 = 2 * g + 1
        psl = pl.ds(g * gsz, gsz)
        pltpu.async_copy(w_ref.at[idx_v.at[cc1]], rows_b, sem_b)
        pltpu.async_copy(p_ref.at[idx_f.at[psl]], pacc_v, sem_p)
        pltpu.make_async_copy(w_ref.at[idx_v.at[cc0]], rows_a, sem_a).wait()
        compute_chunk(cc0, rows_a)

        @pl.when(cc0 + 2 < nch)
        def _():
            pltpu.async_copy(w_ref.at[idx_v.at[cc0 + 2]], rows_a, sem_a)

        pltpu.make_async_copy(w_ref.at[idx_v.at[cc1]], rows_b, sem_b).wait()
        compute_chunk(cc1, rows_b)
        pltpu.make_async_copy(p_ref.at[idx_f.at[psl]], pacc_v, sem_p).wait()
        for r in range(gsz):
            acc = acc + pacc_v[r, pl.ds(0, _LANES)]
        return acc

    acc = jax.lax.fori_loop(0, nch // 2, body,
                            jnp.zeros((_LANES,), jnp.float32))
    aux_v[...] = acc
    pltpu.sync_copy(aux_v, aux_ref.at[wid])


def _emit_kernel(w_ref, idx_ref, c_ref, out_ref):
    j = pl.program_id(1)
    t = idx_ref.shape[0]
    nblk = w_ref.shape[0]
    iota = jax.lax.broadcasted_iota(jnp.int32, (t, nblk), 1) + j * nblk
    gated = jnp.zeros((t, nblk), jnp.float32)
    for k in range(_K):
        ik = idx_ref[:, k:k + 1]
        ck = c_ref[:, k:k + 1]
        gated = gated + jnp.where(iota == ik, ck, 0.0)
    partial = jax.lax.dot_general(
        gated, w_ref[...], (((1,), (0,)), ((), ())),
        preferred_element_type=jnp.float32)

    @pl.when(j == 0)
    def _():
        out_ref[...] = jnp.zeros_like(out_ref)
    out_ref[...] += partial


def kernel(x, know_emb, know_w, router_w, attention_mask):
    b, s, d = x.shape
    n = router_w.shape[1]
    tokens = b * s
    t_blk = min(512, tokens)
    nblk = min(1024, n)
    gi, gj = tokens // t_blk, n // nblk
    xf = x.reshape(tokens, d)

    idx, c, psum = pl.pallas_call(
        functools.partial(_router_kernel, gj),
        grid=(gi, gj),
        in_specs=[
            pl.BlockSpec((t_blk, d), lambda i, j: (i, 0)),
            pl.BlockSpec((d, nblk), lambda i, j: (0, j)),
            pl.BlockSpec((nblk, d), lambda i, j: (j, 0)),
        ],
        out_specs=[
            pl.BlockSpec((t_blk, _K), lambda i, j: (i, 0)),
            pl.BlockSpec((t_blk, _K), lambda i, j: (i, 0)),
            pl.BlockSpec((n, _PW), lambda i, j: (0, 0)),
        ],
        out_shape=[
            jax.ShapeDtypeStruct((tokens, _K), jnp.int32),
            jax.ShapeDtypeStruct((tokens, _K), jnp.float32),
            jax.ShapeDtypeStruct((n, _PW), jnp.float32),
        ],
        scratch_shapes=[
            pltpu.VMEM((gj, t_blk, nblk), jnp.float32),
            pltpu.VMEM((gj, t_blk, _K), jnp.float32),
            pltpu.VMEM((gj, t_blk, _K), jnp.int32),
            pltpu.VMEM((gj, t_blk, _K), jnp.float32),
        ],
    )(xf, router_w, know_emb)

    nw = _NC * _NS
    tpw = tokens // nw
    cpt = 4
    nch = tpw // cpt
    idx3 = idx.reshape(nw, nch, cpt * _K)
    idxf = idx.reshape(nw, tpw * _K)
    c2 = jnp.broadcast_to(
        c.reshape(nw, tpw * _K, 1),
        (nw, tpw * _K, _LANES)).reshape(nw, tpw * _K * _LANES)
    mesh = plsc.VectorSubcoreMesh(core_axis_name="c", subcore_axis_name="s")
    out, aux_p = pl.kernel(
        functools.partial(_sc_emit_kernel, nch, cpt, d),
        mesh=mesh,
        out_type=[
            jax.ShapeDtypeStruct((tokens, d), jnp.float32),
            jax.ShapeDtypeStruct((nw, _LANES), jnp.float32),
        ],
        scratch_types=[
            pltpu.VMEM((nch, cpt * _K), jnp.int32),
            pltpu.VMEM((tpw * _K,), jnp.int32),
            pltpu.VMEM((tpw * _K * _LANES,), jnp.float32),
            pltpu.VMEM((cpt * _K, d), jnp.float32),
            pltpu.VMEM((cpt * _K, d), jnp.float32),
            pltpu.VMEM((cpt, d), jnp.float32),
            pltpu.VMEM((2 * cpt * _K, _PW), jnp.float32),
            pltpu.VMEM((_LANES,), jnp.float32),
            pltpu.SemaphoreType.DMA,
            pltpu.SemaphoreType.DMA,
            pltpu.SemaphoreType.DMA,
        ],
    )(know_w, idx3, idxf, c2, psum)

    aux = (jnp.float32(n) / (jnp.float32(tokens) * jnp.float32(tokens * _K))
           ) * jnp.sum(aux_p[:, 0])
    return out.reshape(b, s, d), aux
